# baseline (speedup 1.0000x reference)
import dataclasses
import functools

import jax
from jax import lax
import jax.numpy as jnp
from jax.experimental import pallas as pl
from jax.experimental.pallas import tpu as pltpu
from jax.experimental.pallas import tpu_sc as plsc

_CS_BLK = 512

_NC, _NS, _L = 2, 16, 16
_NW = _NC * _NS

_CC = 16


def _gating_kernel(x_ref, wg_ref, u_ref, sslot_ref, gslot_ref,
                   g1e_ref, g2e_ref, laux_ref, xpk_ref, *, S, E, CAP):
    logits = jax.lax.dot_general(
        wg_ref[...], x_ref[...],
        dimension_numbers=(((0,), (1,)), ((), ())),
        preferred_element_type=jnp.float32,
    )

    m = jnp.max(logits, axis=0, keepdims=True)
    ex = jnp.exp(logits - m)
    gates = ex / jnp.sum(ex, axis=0, keepdims=True)

    iota = jax.lax.broadcasted_iota(jnp.int32, (E, S), 0)
    big = jnp.int32(E)

    gmax = jnp.max(gates, axis=0, keepdims=True)
    idx1 = jnp.min(jnp.where(gates == gmax, iota, big), axis=0, keepdims=True)
    mask1 = (iota == idx1).astype(jnp.float32)

    neg_inf = jnp.float32(-jnp.inf)
    logits2 = jnp.where(mask1 > 0, neg_inf, logits)
    lmax = jnp.max(logits2, axis=0, keepdims=True)
    idx2 = jnp.min(jnp.where(logits2 == lmax, iota, big), axis=0, keepdims=True)
    mask2 = (iota == idx2).astype(jnp.float32)

    mm = jnp.concatenate([mask1, mask2], axis=0)
    U = u_ref[...]
    R = U.shape[0]
    carry = jnp.zeros((2 * E, 1), jnp.float32)
    blocks = []
    for i in range(S // R):
        blk = mm[:, i * R:(i + 1) * R]
        csb = jax.lax.dot_general(
            blk, U, dimension_numbers=(((1,), (0,)), ((), ())),
            preferred_element_type=jnp.float32,
        ) + carry
        blocks.append(csb)
        carry = csb[:, R - 1:R]
    cs = jnp.concatenate(blocks, axis=1)

    count1 = carry[:E]
    loc1 = cs[:E] - 1.0
    loc2 = cs[E:] - 1.0 + count1

    me = jnp.mean(gates, axis=1, keepdims=True)
    ce = jnp.mean(mask1, axis=1, keepdims=True)
    laux_ref[...] = jnp.sum(me * ce, axis=0, keepdims=True) * jnp.float32(E)

    cap = jnp.float32(CAP)
    mask1k = mask1 * (loc1 < cap).astype(jnp.float32)
    mask2k = mask2 * (loc2 < cap).astype(jnp.float32)

    g1 = jnp.sum(gates * mask1k, axis=0, keepdims=True)
    g2 = jnp.sum(gates * mask2k, axis=0, keepdims=True)
    denom = jnp.maximum(g1 + g2, jnp.float32(1e-9))
    g1 = g1 / denom
    g2 = g2 / denom

    kept1 = jnp.sum(mask1k, axis=0, keepdims=True)
    kept2 = jnp.sum(mask2k, axis=0, keepdims=True)
    loc1_i = jnp.sum(loc1 * mask1k, axis=0, keepdims=True).astype(jnp.int32)
    loc2_i = jnp.sum(loc2 * mask2k, axis=0, keepdims=True).astype(jnp.int32)

    capi = jnp.int32(CAP)
    gs1 = idx1 * capi + loc1_i
    gs2 = idx2 * capi + loc2_i

    tok = jax.lax.broadcasted_iota(jnp.int32, (1, S), 1)
    dump = jnp.int32(E * CAP) + 2 * jax.lax.shift_right_logical(tok, 7)
    ss1 = jnp.where(kept1 > 0, gs1, dump)
    ss2 = jnp.where(kept2 > 0, gs2, dump + 1)

    sslot_ref[...] = jnp.concatenate([ss1, ss2], axis=0)
    gslot_ref[...] = jnp.concatenate([gs1, gs2], axis=0)

    ones_row = jnp.full((1, _L), 1.0, jnp.float32)
    g1e_ref[...] = jax.lax.dot_general(
        g1, ones_row, dimension_numbers=(((0,), (0,)), ((), ())),
        preferred_element_type=jnp.float32,
        precision=jax.lax.Precision.HIGHEST,
    )
    g2e_ref[...] = jax.lax.dot_general(
        g2, ones_row, dimension_numbers=(((0,), (0,)), ((), ())),
        preferred_element_type=jnp.float32,
        precision=jax.lax.Precision.HIGHEST,
    )

    Dh = x_ref.shape[1] // 2
    RB = S // 4
    for i in range(4):
        u = jax.lax.bitcast_convert_type(
            x_ref[i * RB:(i + 1) * RB, :], jnp.uint32)
        rnd = (u + jnp.uint32(0x7FFF) +
               (jax.lax.shift_right_logical(u, jnp.uint32(16)) & jnp.uint32(1)))
        rnd = rnd & jnp.uint32(0xFFFF0000)
        word = (jax.lax.shift_right_logical(rnd[:, :Dh], jnp.uint32(16))
                | rnd[:, Dh:])
        xpk_ref[i * RB:(i + 1) * RB, :] = jax.lax.bitcast_convert_type(
            word, jnp.float32)


def _expert_kernel(disp_ref, wa_ref, wb_ref, b_ref, o_ref):
    u = jax.lax.bitcast_convert_type(disp_ref[...], jnp.uint32)
    xa = jax.lax.bitcast_convert_type(
        jax.lax.shift_left(u, jnp.uint32(16)), jnp.float32
    ).astype(jnp.bfloat16)
    xb = jax.lax.bitcast_convert_type(
        u & jnp.uint32(0xFFFF0000), jnp.float32
    ).astype(jnp.bfloat16)
    wa = wa_ref[0].astype(jnp.bfloat16)
    wb = wb_ref[0].astype(jnp.bfloat16)
    y = jax.lax.dot_general(
        xa, wa, dimension_numbers=(((1,), (0,)), ((), ())),
        preferred_element_type=jnp.float32,
    ) + jax.lax.dot_general(
        xb, wb, dimension_numbers=(((1,), (0,)), ((), ())),
        preferred_element_type=jnp.float32,
    ) + b_ref[0]
    uy = jax.lax.bitcast_convert_type(y, jnp.uint32)
    rnd = (uy + jnp.uint32(0x7FFF) +
           (jax.lax.shift_right_logical(uy, jnp.uint32(16)) & jnp.uint32(1)))
    rnd = rnd & jnp.uint32(0xFFFF0000)
    Dh2 = y.shape[1] // 2
    word = (jax.lax.shift_right_logical(rnd[:, :Dh2], jnp.uint32(16))
            | rnd[:, Dh2:])
    o_ref[...] = jax.lax.bitcast_convert_type(word, jnp.float32)


def _dispatch_body(x_hbm, ss_hbm, disp_hbm, xa, xb, i1a, i2a, i1b, i2b,
                   s1a, s2a, s1b, s2b, *, tok_per_w):
    wid = lax.axis_index("s") * _NC + lax.axis_index("c")
    base = wid * tok_per_w
    nch = 4
    ch = tok_per_w // nch
    bufs = [(xa, i1a, i2a, s1a, s2a), (xb, i1b, i2b, s1b, s2b)]
    pending = [None, None]
    for ci in range(nch):
        xv, i1, i2, s1, s2 = bufs[ci % 2]
        if pending[ci % 2] is not None:
            c1, c2 = pending[ci % 2]
            c1.wait()
            c2.wait()
        tb = base + ci * ch
        pltpu.sync_copy(ss_hbm.at[0, pl.ds(tb, ch)], i1)
        pltpu.sync_copy(ss_hbm.at[1, pl.ds(tb, ch)], i2)
        pltpu.sync_copy(x_hbm.at[pl.ds(tb, ch)], xv)
        c1 = pltpu.async_copy(xv, disp_hbm.at[i1], s1)
        c2 = pltpu.async_copy(xv, disp_hbm.at[i2], s2)
        pending[ci % 2] = (c1, c2)
    for c1, c2 in pending:
        c1.wait()
        c2.wait()


def _combine_body(eo_hbm, gs_hbm, g1e_hbm, g2e_hbm, out_hbm,
                  i1v, i2v, g1v, g2v, r1a, r2a, r1b, r2b, ov,
                  s1a, s2a, s1b, s2b, *, tok_per_w, D):
    wid = lax.axis_index("s") * _NC + lax.axis_index("c")
    base = wid * tok_per_w
    nch = tok_per_w // _CC

    pltpu.sync_copy(gs_hbm.at[0, pl.ds(base, tok_per_w)], i1v)
    pltpu.sync_copy(gs_hbm.at[1, pl.ds(base, tok_per_w)], i2v)
    pltpu.sync_copy(g1e_hbm.at[pl.ds(base, tok_per_w)], g1v)
    pltpu.sync_copy(g2e_hbm.at[pl.ds(base, tok_per_w)], g2v)

    def _start(ci, r1, r2, s1, s2):
        pltpu.async_copy(eo_hbm.at[i1v.at[pl.ds(ci * _CC, _CC)]], r1, s1)
        pltpu.async_copy(eo_hbm.at[i2v.at[pl.ds(ci * _CC, _CC)]], r2, s2)

    def _wait(r1, r2, s1, s2):
        pltpu.make_async_copy(eo_hbm.at[i1v.at[pl.ds(0, _CC)]], r1, s1).wait()
        pltpu.make_async_copy(eo_hbm.at[i2v.at[pl.ds(0, _CC)]], r2, s2).wait()

    Dh = D // 2
    shift = jnp.uint32(16)
    hmask = jnp.uint32(0xFFFF0000)

    def _compute_store(ci, r1, r2):
        @pl.loop(0, _CC)
        def _tok(t):
            g1b = g1v[ci * _CC + t]
            g2b = g2v[ci * _CC + t]
            for j in range(Dh // _L):
                sl = pl.ds(j * _L, _L)
                w1 = plsc.bitcast(r1[t, sl], jnp.uint32)
                w2 = plsc.bitcast(r2[t, sl], jnp.uint32)
                lo1 = plsc.bitcast(jax.lax.shift_left(w1, shift), jnp.float32)
                lo2 = plsc.bitcast(jax.lax.shift_left(w2, shift), jnp.float32)
                hi1 = plsc.bitcast(w1 & hmask, jnp.float32)
                hi2 = plsc.bitcast(w2 & hmask, jnp.float32)
                ov[t, sl] = g1b * lo1 + g2b * lo2
                ov[t, pl.ds(Dh + j * _L, _L)] = g1b * hi1 + g2b * hi2

        pltpu.sync_copy(ov, out_hbm.at[pl.ds(base + ci * _CC, _CC)])

    _start(0, r1a, r2a, s1a, s2a)

    @pl.loop(0, nch, step=2)
    def _pair(ci):
        _start(ci + 1, r1b, r2b, s1b, s2b)
        _wait(r1a, r2a, s1a, s2a)
        _compute_store(ci, r1a, r2a)

        @pl.when(ci + 2 < nch)
        def _():
            _start(ci + 2, r1a, r2a, s1a, s2a)

        _wait(r1b, r2b, s1b, s2b)
        _compute_store(ci + 1, r1b, r2b)


def kernel(x, wg, W, b):
    B, T, D = x.shape
    E = wg.shape[1]
    S = B * T
    CAP = 2 * S // E
    R = _CS_BLK
    tok_per_w = S // _NW

    xr = x.reshape(S, D)
    U = jnp.triu(jnp.ones((R, R), jnp.float32))

    sslots, gslots, g1e, g2e, laux, xpk = pl.pallas_call(
        functools.partial(_gating_kernel, S=S, E=E, CAP=CAP),
        out_shape=[
            jax.ShapeDtypeStruct((2, S), jnp.int32),
            jax.ShapeDtypeStruct((2, S), jnp.int32),
            jax.ShapeDtypeStruct((S, _L), jnp.float32),
            jax.ShapeDtypeStruct((S, _L), jnp.float32),
            jax.ShapeDtypeStruct((1, 1), jnp.float32),
            jax.ShapeDtypeStruct((S, D // 2), jnp.float32),
        ],
    )(xr, wg, U)

    mesh = plsc.VectorSubcoreMesh(core_axis_name="c", subcore_axis_name="s")
    cp = pltpu.CompilerParams()
    if "needs_layout_passes" in pltpu.CompilerParams.__dataclass_fields__:
        cp = dataclasses.replace(cp, needs_layout_passes=False)

    dispatch = pl.kernel(
        functools.partial(_dispatch_body, tok_per_w=tok_per_w),
        out_type=jax.ShapeDtypeStruct((E * CAP + CAP, D // 2), jnp.float32),
        mesh=mesh,
        scratch_types=[
            pltpu.VMEM((tok_per_w // 4, D // 2), jnp.float32),
            pltpu.VMEM((tok_per_w // 4, D // 2), jnp.float32),
            pltpu.VMEM((tok_per_w // 4,), jnp.int32),
            pltpu.VMEM((tok_per_w // 4,), jnp.int32),
            pltpu.VMEM((tok_per_w // 4,), jnp.int32),
            pltpu.VMEM((tok_per_w // 4,), jnp.int32),
            pltpu.SemaphoreType.DMA,
            pltpu.SemaphoreType.DMA,
            pltpu.SemaphoreType.DMA,
            pltpu.SemaphoreType.DMA,
        ],
    )
    disp = dispatch(xpk, sslots)

    b3 = b.reshape(E, 1, D)

    eo = pl.pallas_call(
        _expert_kernel,
        grid=(E,),
        in_specs=[
            pl.BlockSpec((CAP, D // 2), lambda e: (e, 0)),
            pl.BlockSpec((1, D // 2, D), lambda e: (e, 0, 0)),
            pl.BlockSpec((1, D // 2, D), lambda e: (e, 1, 0)),
            pl.BlockSpec((1, 1, D), lambda e: (e, 0, 0)),
        ],
        out_specs=pl.BlockSpec((CAP, D // 2), lambda e: (e, 0)),
        out_shape=jax.ShapeDtypeStruct((E * CAP, D // 2), jnp.float32),
    )(disp, W, W, b3)

    combine = pl.kernel(
        functools.partial(_combine_body, tok_per_w=tok_per_w, D=D),
        out_type=jax.ShapeDtypeStruct((S, D), jnp.float32),
        mesh=mesh,
        compiler_params=cp,
        scratch_types=[
            pltpu.VMEM((tok_per_w,), jnp.int32),
            pltpu.VMEM((tok_per_w,), jnp.int32),
            pltpu.VMEM((tok_per_w, _L), jnp.float32),
            pltpu.VMEM((tok_per_w, _L), jnp.float32),
            pltpu.VMEM((_CC, D // 2), jnp.float32),
            pltpu.VMEM((_CC, D // 2), jnp.float32),
            pltpu.VMEM((_CC, D // 2), jnp.float32),
            pltpu.VMEM((_CC, D // 2), jnp.float32),
            pltpu.VMEM((_CC, D), jnp.float32),
            pltpu.SemaphoreType.DMA,
            pltpu.SemaphoreType.DMA,
            pltpu.SemaphoreType.DMA,
            pltpu.SemaphoreType.DMA,
        ],
    )
    out = combine(eo, gslots, g1e, g2e)

    return out.reshape(B, T, D), laux[0, 0]


# speedup vs baseline: 1.0993x; 1.0993x over previous
import dataclasses
import functools

import jax
from jax import lax
import jax.numpy as jnp
from jax.experimental import pallas as pl
from jax.experimental.pallas import tpu as pltpu
from jax.experimental.pallas import tpu_sc as plsc

_CS_BLK = 512

_NC, _NS, _L = 2, 16, 16
_NW = _NC * _NS

_CC = 16


def _gating_kernel(x_ref, wg_ref, u_ref, sslot_ref, gslot_ref,
                   g1e_ref, g2e_ref, laux_ref, xpk_ref, *, S, E, CAP):
    logits = jax.lax.dot_general(
        wg_ref[...], x_ref[...],
        dimension_numbers=(((0,), (1,)), ((), ())),
        preferred_element_type=jnp.float32,
    )

    m = jnp.max(logits, axis=0, keepdims=True)
    ex = jnp.exp(logits - m)
    gates = ex / jnp.sum(ex, axis=0, keepdims=True)

    iota = jax.lax.broadcasted_iota(jnp.int32, (E, S), 0)
    big = jnp.int32(E)

    gmax = jnp.max(gates, axis=0, keepdims=True)
    idx1 = jnp.min(jnp.where(gates == gmax, iota, big), axis=0, keepdims=True)
    mask1 = (iota == idx1).astype(jnp.float32)

    neg_inf = jnp.float32(-jnp.inf)
    logits2 = jnp.where(mask1 > 0, neg_inf, logits)
    lmax = jnp.max(logits2, axis=0, keepdims=True)
    idx2 = jnp.min(jnp.where(logits2 == lmax, iota, big), axis=0, keepdims=True)
    mask2 = (iota == idx2).astype(jnp.float32)

    mm = jnp.concatenate([mask1, mask2], axis=0)
    U = u_ref[...]
    R = U.shape[0]
    carry = jnp.zeros((2 * E, 1), jnp.float32)
    blocks = []
    for i in range(S // R):
        blk = mm[:, i * R:(i + 1) * R]
        csb = jax.lax.dot_general(
            blk, U, dimension_numbers=(((1,), (0,)), ((), ())),
            preferred_element_type=jnp.float32,
        ) + carry
        blocks.append(csb)
        carry = csb[:, R - 1:R]
    cs = jnp.concatenate(blocks, axis=1)

    count1 = carry[:E]
    loc1 = cs[:E] - 1.0
    loc2 = cs[E:] - 1.0 + count1

    me = jnp.mean(gates, axis=1, keepdims=True)
    ce = jnp.mean(mask1, axis=1, keepdims=True)
    laux_ref[...] = jnp.sum(me * ce, axis=0, keepdims=True) * jnp.float32(E)

    cap = jnp.float32(CAP)
    mask1k = mask1 * (loc1 < cap).astype(jnp.float32)
    mask2k = mask2 * (loc2 < cap).astype(jnp.float32)

    g1 = jnp.sum(gates * mask1k, axis=0, keepdims=True)
    g2 = jnp.sum(gates * mask2k, axis=0, keepdims=True)
    denom = jnp.maximum(g1 + g2, jnp.float32(1e-9))
    g1 = g1 / denom
    g2 = g2 / denom

    kept1 = jnp.sum(mask1k, axis=0, keepdims=True)
    kept2 = jnp.sum(mask2k, axis=0, keepdims=True)
    loc1_i = jnp.sum(loc1 * mask1k, axis=0, keepdims=True).astype(jnp.int32)
    loc2_i = jnp.sum(loc2 * mask2k, axis=0, keepdims=True).astype(jnp.int32)

    capi = jnp.int32(CAP)
    gs1 = idx1 * capi + loc1_i
    gs2 = idx2 * capi + loc2_i

    tok = jax.lax.broadcasted_iota(jnp.int32, (1, S), 1)
    dump = jnp.int32(E * CAP) + 2 * jax.lax.shift_right_logical(tok, 7)
    ss1 = jnp.where(kept1 > 0, gs1, dump)
    ss2 = jnp.where(kept2 > 0, gs2, dump + 1)

    sslot_ref[...] = jnp.concatenate([ss1, ss2], axis=0)
    gslot_ref[...] = jnp.concatenate([gs1, gs2], axis=0)

    ones_row = jnp.full((1, _L), 1.0, jnp.float32)
    g1e_ref[...] = jax.lax.dot_general(
        g1, ones_row, dimension_numbers=(((0,), (0,)), ((), ())),
        preferred_element_type=jnp.float32,
        precision=jax.lax.Precision.HIGHEST,
    )
    g2e_ref[...] = jax.lax.dot_general(
        g2, ones_row, dimension_numbers=(((0,), (0,)), ((), ())),
        preferred_element_type=jnp.float32,
        precision=jax.lax.Precision.HIGHEST,
    )

    Dh = x_ref.shape[1] // 2
    RB = S // 4
    for i in range(4):
        u = jax.lax.bitcast_convert_type(
            x_ref[i * RB:(i + 1) * RB, :], jnp.uint32)
        rnd = (u + jnp.uint32(0x7FFF) +
               (jax.lax.shift_right_logical(u, jnp.uint32(16)) & jnp.uint32(1)))
        rnd = rnd & jnp.uint32(0xFFFF0000)
        word = (jax.lax.shift_right_logical(rnd[:, :Dh], jnp.uint32(16))
                | rnd[:, Dh:])
        xpk_ref[i * RB:(i + 1) * RB, :] = jax.lax.bitcast_convert_type(
            word, jnp.float32)


def _expert_kernel(disp_ref, wa_ref, wb_ref, b_ref, o_ref):
    u = jax.lax.bitcast_convert_type(disp_ref[...], jnp.uint32)
    xa = jax.lax.bitcast_convert_type(
        jax.lax.shift_left(u, jnp.uint32(16)), jnp.float32
    ).astype(jnp.bfloat16)
    xb = jax.lax.bitcast_convert_type(
        u & jnp.uint32(0xFFFF0000), jnp.float32
    ).astype(jnp.bfloat16)
    wa = wa_ref[0].astype(jnp.bfloat16)
    wb = wb_ref[0].astype(jnp.bfloat16)
    y = jax.lax.dot_general(
        xa, wa, dimension_numbers=(((1,), (0,)), ((), ())),
        preferred_element_type=jnp.float32,
    ) + jax.lax.dot_general(
        xb, wb, dimension_numbers=(((1,), (0,)), ((), ())),
        preferred_element_type=jnp.float32,
    )
    o_ref[...] = y + b_ref[0]


def _dispatch_body(x_hbm, ss_hbm, disp_hbm, xa, xb, i1a, i2a, i1b, i2b,
                   s1a, s2a, s1b, s2b, *, tok_per_w):
    wid = lax.axis_index("s") * _NC + lax.axis_index("c")
    base = wid * tok_per_w
    nch = 4
    ch = tok_per_w // nch
    bufs = [(xa, i1a, i2a, s1a, s2a), (xb, i1b, i2b, s1b, s2b)]
    pending = [None, None]
    for ci in range(nch):
        xv, i1, i2, s1, s2 = bufs[ci % 2]
        if pending[ci % 2] is not None:
            c1, c2 = pending[ci % 2]
            c1.wait()
            c2.wait()
        tb = base + ci * ch
        pltpu.sync_copy(ss_hbm.at[0, pl.ds(tb, ch)], i1)
        pltpu.sync_copy(ss_hbm.at[1, pl.ds(tb, ch)], i2)
        pltpu.sync_copy(x_hbm.at[pl.ds(tb, ch)], xv)
        c1 = pltpu.async_copy(xv, disp_hbm.at[i1], s1)
        c2 = pltpu.async_copy(xv, disp_hbm.at[i2], s2)
        pending[ci % 2] = (c1, c2)
    for c1, c2 in pending:
        c1.wait()
        c2.wait()


def _combine_body(eo_hbm, gs_hbm, g1e_hbm, g2e_hbm, out_hbm,
                  i1v, i2v, g1v, g2v, r1a, r2a, r1b, r2b, ov,
                  s1a, s2a, s1b, s2b, *, tok_per_w, D):
    wid = lax.axis_index("s") * _NC + lax.axis_index("c")
    base = wid * tok_per_w
    nch = tok_per_w // _CC

    pltpu.sync_copy(gs_hbm.at[0, pl.ds(base, tok_per_w)], i1v)
    pltpu.sync_copy(gs_hbm.at[1, pl.ds(base, tok_per_w)], i2v)
    pltpu.sync_copy(g1e_hbm.at[pl.ds(base, tok_per_w)], g1v)
    pltpu.sync_copy(g2e_hbm.at[pl.ds(base, tok_per_w)], g2v)

    def _start(ci, r1, r2, s1, s2):
        pltpu.async_copy(eo_hbm.at[i1v.at[pl.ds(ci * _CC, _CC)]], r1, s1)
        pltpu.async_copy(eo_hbm.at[i2v.at[pl.ds(ci * _CC, _CC)]], r2, s2)

    def _wait(r1, r2, s1, s2):
        pltpu.make_async_copy(eo_hbm.at[i1v.at[pl.ds(0, _CC)]], r1, s1).wait()
        pltpu.make_async_copy(eo_hbm.at[i2v.at[pl.ds(0, _CC)]], r2, s2).wait()

    def _compute_store(ci, r1, r2):
        @pl.loop(0, _CC)
        def _tok(t):
            g1b = g1v[ci * _CC + t]
            g2b = g2v[ci * _CC + t]
            for j in range(D // _L):
                sl = pl.ds(j * _L, _L)
                ov[t, sl] = g1b * r1[t, sl] + g2b * r2[t, sl]

        pltpu.sync_copy(ov, out_hbm.at[pl.ds(base + ci * _CC, _CC)])

    _start(0, r1a, r2a, s1a, s2a)

    @pl.loop(0, nch, step=2)
    def _pair(ci):
        _start(ci + 1, r1b, r2b, s1b, s2b)
        _wait(r1a, r2a, s1a, s2a)
        _compute_store(ci, r1a, r2a)

        @pl.when(ci + 2 < nch)
        def _():
            _start(ci + 2, r1a, r2a, s1a, s2a)

        _wait(r1b, r2b, s1b, s2b)
        _compute_store(ci + 1, r1b, r2b)


def kernel(x, wg, W, b):
    B, T, D = x.shape
    E = wg.shape[1]
    S = B * T
    CAP = 2 * S // E
    R = _CS_BLK
    tok_per_w = S // _NW

    xr = x.reshape(S, D)
    U = jnp.triu(jnp.ones((R, R), jnp.float32))

    sslots, gslots, g1e, g2e, laux, xpk = pl.pallas_call(
        functools.partial(_gating_kernel, S=S, E=E, CAP=CAP),
        out_shape=[
            jax.ShapeDtypeStruct((2, S), jnp.int32),
            jax.ShapeDtypeStruct((2, S), jnp.int32),
            jax.ShapeDtypeStruct((S, _L), jnp.float32),
            jax.ShapeDtypeStruct((S, _L), jnp.float32),
            jax.ShapeDtypeStruct((1, 1), jnp.float32),
            jax.ShapeDtypeStruct((S, D // 2), jnp.float32),
        ],
    )(xr, wg, U)

    mesh = plsc.VectorSubcoreMesh(core_axis_name="c", subcore_axis_name="s")
    cp = pltpu.CompilerParams()
    if "needs_layout_passes" in pltpu.CompilerParams.__dataclass_fields__:
        cp = dataclasses.replace(cp, needs_layout_passes=False)

    dispatch = pl.kernel(
        functools.partial(_dispatch_body, tok_per_w=tok_per_w),
        out_type=jax.ShapeDtypeStruct((E * CAP + CAP, D // 2), jnp.float32),
        mesh=mesh,
        scratch_types=[
            pltpu.VMEM((tok_per_w // 4, D // 2), jnp.float32),
            pltpu.VMEM((tok_per_w // 4, D // 2), jnp.float32),
            pltpu.VMEM((tok_per_w // 4,), jnp.int32),
            pltpu.VMEM((tok_per_w // 4,), jnp.int32),
            pltpu.VMEM((tok_per_w // 4,), jnp.int32),
            pltpu.VMEM((tok_per_w // 4,), jnp.int32),
            pltpu.SemaphoreType.DMA,
            pltpu.SemaphoreType.DMA,
            pltpu.SemaphoreType.DMA,
            pltpu.SemaphoreType.DMA,
        ],
    )
    disp = dispatch(xpk, sslots)

    b3 = b.reshape(E, 1, D)

    eo = pl.pallas_call(
        _expert_kernel,
        grid=(E,),
        in_specs=[
            pl.BlockSpec((CAP, D // 2), lambda e: (e, 0)),
            pl.BlockSpec((1, D // 2, D), lambda e: (e, 0, 0)),
            pl.BlockSpec((1, D // 2, D), lambda e: (e, 1, 0)),
            pl.BlockSpec((1, 1, D), lambda e: (e, 0, 0)),
        ],
        out_specs=pl.BlockSpec((CAP, D), lambda e: (e, 0)),
        out_shape=jax.ShapeDtypeStruct((E * CAP, D), jnp.float32),
    )(disp, W, W, b3)

    combine = pl.kernel(
        functools.partial(_combine_body, tok_per_w=tok_per_w, D=D),
        out_type=jax.ShapeDtypeStruct((S, D), jnp.float32),
        mesh=mesh,
        compiler_params=cp,
        scratch_types=[
            pltpu.VMEM((tok_per_w,), jnp.int32),
            pltpu.VMEM((tok_per_w,), jnp.int32),
            pltpu.VMEM((tok_per_w, _L), jnp.float32),
            pltpu.VMEM((tok_per_w, _L), jnp.float32),
            pltpu.VMEM((_CC, D), jnp.float32),
            pltpu.VMEM((_CC, D), jnp.float32),
            pltpu.VMEM((_CC, D), jnp.float32),
            pltpu.VMEM((_CC, D), jnp.float32),
            pltpu.VMEM((_CC, D), jnp.float32),
            pltpu.SemaphoreType.DMA,
            pltpu.SemaphoreType.DMA,
            pltpu.SemaphoreType.DMA,
            pltpu.SemaphoreType.DMA,
        ],
    )
    out = combine(eo, gslots, g1e, g2e)

    return out.reshape(B, T, D), laux[0, 0]
